# Initial kernel scaffold; baseline (speedup 1.0000x reference)
import jax
import jax.numpy as jnp
from jax.experimental import pallas as pl


def kernel(x, edge_index, edge_attr, batch, W1, b1, gamma, beta, W2, b2):
    raise NotImplementedError("write your pallas kernel here")


# speedup vs baseline: 1.4258x; 1.4258x over previous
import functools

import jax
import jax.numpy as jnp
from jax import lax
from jax.experimental import pallas as pl
from jax.experimental.pallas import tpu as pltpu
from jax.experimental.pallas import tpu_sc as plsc

N = 10000
D = 128
MID = 128
OUT = 128
E = 320000

NC = 2
NS = 16
NW = NC * NS

B = 128
NBLK = 80
E_PAD = NW * B * NBLK


def _sc_gather_combine(tab, colp, eap):
    mesh = plsc.VectorSubcoreMesh(
        core_axis_name="c", subcore_axis_name="s", num_cores=NC, num_subcores=NS
    )

    @functools.partial(
        pl.kernel,
        out_type=jax.ShapeDtypeStruct((E_PAD, MID), jnp.float32),
        mesh=mesh,
        scratch_types=[
            pltpu.VMEM((B,), jnp.int32),
            pltpu.VMEM((B,), jnp.float32),
            pltpu.VMEM((B, 2 * MID), jnp.float32),
            pltpu.VMEM((B, MID), jnp.float32),
            pltpu.SemaphoreType.DMA,
        ],
    )
    def k(tab_hbm, col_hbm, ea_hbm, u_out, colv, eav, rowsbuf, ubuf, sem):
        c = lax.axis_index("c")
        s = lax.axis_index("s")
        wid = c * NS + s
        ebase = wid * (B * NBLK)

        def blk(i, carry):
            off = ebase + i * B
            pltpu.sync_copy(col_hbm.at[pl.ds(off, B)], colv)
            pltpu.sync_copy(ea_hbm.at[pl.ds(off, B)], eav)
            pltpu.async_copy(tab_hbm.at[colv], rowsbuf, sem).wait()

            def edge16(g, carry2):
                ev16 = eav[pl.ds(16 * g, 16)]
                for l in range(16):
                    b = 16 * g + l
                    ev = jnp.full((16,), ev16[l], jnp.float32)
                    for j in range(MID // 16):
                        y = rowsbuf[b, pl.ds(16 * j, 16)]
                        z = rowsbuf[b, pl.ds(MID + 16 * j, 16)]
                        ubuf[b, pl.ds(16 * j, 16)] = y + ev * z
                return carry2
            lax.fori_loop(0, B // 16, edge16, 0)

            pltpu.sync_copy(ubuf, u_out.at[pl.ds(off, B)])
            return carry
        lax.fori_loop(0, NBLK, blk, 0)

    return k(tab, colp, eap)


def _tc_matmul(x, wcat):
    bn = 2000

    def body(x_ref, w_ref, o_ref):
        o_ref[...] = jnp.dot(x_ref[...], w_ref[...],
                             preferred_element_type=jnp.float32)

    return pl.pallas_call(
        body,
        out_shape=jax.ShapeDtypeStruct((N, 2 * MID), jnp.float32),
        grid=(N // bn,),
        in_specs=[
            pl.BlockSpec((bn, D), lambda i: (i, 0)),
            pl.BlockSpec((D, 2 * MID), lambda i: (0, 0)),
        ],
        out_specs=pl.BlockSpec((bn, 2 * MID), lambda i: (i, 0)),
    )(x, wcat)


def _tc_mean_relu_stats(acc, cnt, b1):
    bn = 2000

    def body(a_ref, cn_ref, b1_ref, h_ref, st_ref):
        c = jnp.maximum(cn_ref[...][:, 0:1], 1.0)
        h = a_ref[...] / c + b1_ref[...]
        hr = jnp.maximum(h, 0.0)
        h_ref[...] = hr

        @pl.when(pl.program_id(0) == 0)
        def _():
            st_ref[...] = jnp.zeros_like(st_ref)

        s1 = jnp.sum(hr, axis=0, keepdims=True)
        s2 = jnp.sum(hr * hr, axis=0, keepdims=True)
        pad = jnp.zeros((6, MID), jnp.float32)
        st_ref[...] += jnp.concatenate([s1, s2, pad], axis=0)

    return pl.pallas_call(
        body,
        out_shape=(
            jax.ShapeDtypeStruct((N, MID), jnp.float32),
            jax.ShapeDtypeStruct((8, MID), jnp.float32),
        ),
        grid=(N // bn,),
        in_specs=[
            pl.BlockSpec((bn, MID), lambda i: (i, 0)),
            pl.BlockSpec((bn, 8), lambda i: (i, 0)),
            pl.BlockSpec((1, MID), lambda i: (0, 0)),
        ],
        out_specs=(
            pl.BlockSpec((bn, MID), lambda i: (i, 0)),
            pl.BlockSpec((8, MID), lambda i: (0, 0)),
        ),
    )(acc, cnt, b1)


def _tc_bn_linear(h, stats, gamma, beta, w2, b2):
    bn = 2000

    def body(h_ref, st_ref, g_ref, be_ref, w2_ref, b2_ref, o_ref):
        hr = h_ref[...]
        st = st_ref[...]
        mu = st[0:1, :] * (1.0 / N)
        var = st[1:2, :] * (1.0 / N) - mu * mu
        rstd = lax.rsqrt(var + 1e-5)
        scale = rstd * g_ref[...]
        shift = be_ref[...] - mu * scale
        xn = hr * scale + shift
        o_ref[...] = jnp.dot(xn, w2_ref[...],
                             preferred_element_type=jnp.float32) + b2_ref[...]

    return pl.pallas_call(
        body,
        out_shape=jax.ShapeDtypeStruct((N, OUT), jnp.float32),
        grid=(N // bn,),
        in_specs=[
            pl.BlockSpec((bn, MID), lambda i: (i, 0)),
            pl.BlockSpec((8, MID), lambda i: (0, 0)),
            pl.BlockSpec((1, MID), lambda i: (0, 0)),
            pl.BlockSpec((1, MID), lambda i: (0, 0)),
            pl.BlockSpec((MID, OUT), lambda i: (0, 0)),
            pl.BlockSpec((1, OUT), lambda i: (0, 0)),
        ],
        out_specs=pl.BlockSpec((bn, OUT), lambda i: (i, 0)),
    )(h, stats, gamma, beta, w2, b2)


def kernel(x, edge_index, edge_attr, batch, W1, b1, gamma, beta, W2, b2):
    del batch
    row = edge_index[0].astype(jnp.int32)
    col = edge_index[1].astype(jnp.int32)
    ea = edge_attr[:, 0].astype(jnp.float32)

    pad = E_PAD - E
    colp = jnp.concatenate([col, jnp.zeros((pad,), jnp.int32)])
    eap = jnp.concatenate([ea, jnp.zeros((pad,), jnp.float32)])

    wcat = jnp.concatenate([W1[:D], W1[D:]], axis=1)

    yz = _tc_matmul(x, wcat)
    u = _sc_gather_combine(yz, colp, eap)[:E]

    acc = jax.ops.segment_sum(u, row, num_segments=N)
    cnt = jax.ops.segment_sum(jnp.ones((E,), jnp.float32), row,
                              num_segments=N)
    cnt8 = jnp.broadcast_to(cnt[:, None], (N, 8))

    h, stats = _tc_mean_relu_stats(acc, cnt8, b1.reshape(1, MID))
    out = _tc_bn_linear(h, stats, gamma.reshape(1, MID), beta.reshape(1, MID),
                        W2, b2.reshape(1, OUT))
    return out
